# baseline (speedup 1.0000x reference)
import functools

import jax
import jax.numpy as jnp
from jax import lax
from jax.experimental import pallas as pl
from jax.experimental.pallas import tpu as pltpu
from jax.experimental.pallas import tpu_sc as plsc

DIM = 64
NC = 2
NS = 16
NW = NC * NS
CHUNK = 1024


@functools.partial(jax.jit, static_argnames=("total",))
def _gather_rows(idx_flat, table, *, total):
    b_per_w = total // NW
    n_chunks = b_per_w // CHUNK
    mesh = plsc.VectorSubcoreMesh(core_axis_name="c", subcore_axis_name="s")

    @functools.partial(
        pl.kernel,
        out_type=jax.ShapeDtypeStruct((total, DIM), jnp.float32),
        mesh=mesh,
        scratch_types=[
            pltpu.VMEM((CHUNK,), jnp.int32),
            pltpu.VMEM((CHUNK, DIM), jnp.float32),
            pltpu.SemaphoreType.DMA,
        ],
        compiler_params=pltpu.CompilerParams(use_tc_tiling_on_sc=False),
    )
    def gather_kernel(idx_hbm, table_hbm, out_hbm, idx_v, rows_v, sem):
        wid = lax.axis_index("s") * NC + lax.axis_index("c")
        base = wid * b_per_w

        def body(i, carry):
            off = base + i * CHUNK
            pltpu.sync_copy(idx_hbm.at[pl.ds(off, CHUNK)], idx_v)
            pltpu.async_copy(table_hbm.at[idx_v], rows_v, sem).wait()
            pltpu.sync_copy(rows_v, out_hbm.at[pl.ds(off, CHUNK)])
            return carry

        lax.fori_loop(0, n_chunks, body, 0)

    return gather_kernel(idx_flat, table)


def kernel(x, emb_weight):
    b, h = x.shape
    total = b * h
    out = _gather_rows(x.reshape(total), emb_weight, total=total)
    return out.reshape(b, h, DIM)


# speedup vs baseline: 1.0142x; 1.0142x over previous
import functools

import jax
import jax.numpy as jnp
from jax import lax
from jax.experimental import pallas as pl
from jax.experimental.pallas import tpu as pltpu
from jax.experimental.pallas import tpu_sc as plsc

DIM = 64
NC = 2
NS = 16
NW = NC * NS
CHUNK = 800


@functools.partial(jax.jit, static_argnames=("total",))
def _gather_rows(idx_flat, table, *, total):
    b_per_w = total // NW
    n_chunks = b_per_w // CHUNK
    n_pairs = n_chunks // 2
    mesh = plsc.VectorSubcoreMesh(core_axis_name="c", subcore_axis_name="s")

    @functools.partial(
        pl.kernel,
        out_type=jax.ShapeDtypeStruct((total, DIM), jnp.float32),
        mesh=mesh,
        scratch_types=[
            pltpu.VMEM((b_per_w,), jnp.int32),
            pltpu.VMEM((CHUNK, DIM), jnp.float32),
            pltpu.VMEM((CHUNK, DIM), jnp.float32),
            pltpu.SemaphoreType.DMA,
            pltpu.SemaphoreType.DMA,
        ],
        compiler_params=pltpu.CompilerParams(use_tc_tiling_on_sc=False),
    )
    def gather_kernel(idx_hbm, table_hbm, out_hbm, idx_v, buf0, buf1,
                      sem0, sem1):
        wid = lax.axis_index("s") * NC + lax.axis_index("c")
        base = wid * b_per_w
        pltpu.sync_copy(idx_hbm.at[pl.ds(base, b_per_w)], idx_v)

        def gather(local_off, buf, sem):
            pltpu.async_copy(
                table_hbm.at[idx_v.at[pl.ds(local_off, CHUNK)]], buf, sem)

        def wait(buf, sem):
            pltpu.make_async_copy(table_hbm.at[pl.ds(0, CHUNK)], buf,
                                  sem).wait()

        gather(0, buf0, sem0)

        def body(j, carry):
            c0 = 2 * j * CHUNK
            gather(c0 + CHUNK, buf1, sem1)
            wait(buf0, sem0)
            pltpu.sync_copy(buf0, out_hbm.at[pl.ds(base + c0, CHUNK)])

            @pl.when(j + 1 < n_pairs)
            def _():
                gather(c0 + 2 * CHUNK, buf0, sem0)

            wait(buf1, sem1)
            pltpu.sync_copy(buf1,
                            out_hbm.at[pl.ds(base + c0 + CHUNK, CHUNK)])
            return carry

        lax.fori_loop(0, n_pairs, body, 0)

    return gather_kernel(idx_flat, table)


def kernel(x, emb_weight):
    b, h = x.shape
    total = b * h
    out = _gather_rows(x.reshape(total), emb_weight, total=total)
    return out.reshape(b, h, DIM)
